# baseline (device time: 25857 ns/iter reference)
import jax
import jax.numpy as jnp
from jax import lax
from jax.experimental import pallas as pl
from jax.experimental.pallas import tpu as pltpu

CHUNK_ROWS = [64] * 6 + [24] * 4 + [8] * 4
N_CHUNKS = len(CHUNK_ROWS)
CHUNK_OFF = [sum(CHUNK_ROWS[:c]) for c in range(N_CHUNKS)]


def kernel(x):
    m, n = x.shape
    half = m // 2
    assert sum(CHUNK_ROWS) == half

    def body(x_hbm, out_hbm, xv, acc, recv_buf,
             lin_sem, lout_sem, s1, r1, s2, r2):
        my_x = lax.axis_index("x")
        my_y = lax.axis_index("y")
        x_nbr = (1 - my_x, my_y)
        y_nbr = (my_x, 1 - my_y)

        row0 = my_y * half

        lin = pltpu.make_async_copy(
            x_hbm.at[pl.ds(row0, half)], xv, lin_sem
        )
        lin.start()

        barrier_sem = pltpu.get_barrier_semaphore()
        for nbr in (x_nbr, y_nbr):
            pl.semaphore_signal(
                barrier_sem, inc=1,
                device_id=nbr, device_id_type=pl.DeviceIdType.MESH,
            )
        pl.semaphore_wait(barrier_sem, 2)

        rdma1 = []
        for c in range(N_CHUNKS):
            rd = pltpu.make_async_remote_copy(
                src_ref=x_hbm.at[pl.ds(row0 + CHUNK_OFF[c], CHUNK_ROWS[c])],
                dst_ref=recv_buf.at[pl.ds(CHUNK_OFF[c], CHUNK_ROWS[c])],
                send_sem=s1.at[c],
                recv_sem=r1.at[c],
                device_id=x_nbr,
                device_id_type=pl.DeviceIdType.MESH,
            )
            rd.start()
            rdma1.append(rd)

        lin.wait()

        rdma2 = []
        for c in range(N_CHUNKS):
            rdma1[c].wait_recv()
            rows = pl.ds(CHUNK_OFF[c], CHUNK_ROWS[c])
            acc[rows, :] = xv[rows, :] + recv_buf[rows, :]
            rd = pltpu.make_async_remote_copy(
                src_ref=acc.at[rows],
                dst_ref=out_hbm.at[pl.ds(row0 + CHUNK_OFF[c], CHUNK_ROWS[c])],
                send_sem=s2.at[c],
                recv_sem=r2.at[c],
                device_id=y_nbr,
                device_id_type=pl.DeviceIdType.MESH,
            )
            rd.start()
            rdma2.append(rd)

        for c in range(N_CHUNKS):
            rdma1[c].wait_send()

        lout = pltpu.make_async_copy(
            acc, out_hbm.at[pl.ds(row0, half)], lout_sem
        )
        lout.start()

        for c in range(N_CHUNKS):
            rdma2[c].wait()
        lout.wait()

    return pl.pallas_call(
        body,
        out_shape=jax.ShapeDtypeStruct((m, n), x.dtype),
        in_specs=[pl.BlockSpec(memory_space=pltpu.MemorySpace.HBM)],
        out_specs=pl.BlockSpec(memory_space=pltpu.MemorySpace.HBM),
        scratch_shapes=[
            pltpu.VMEM((half, n), x.dtype),
            pltpu.VMEM((half, n), x.dtype),
            pltpu.VMEM((half, n), x.dtype),
            pltpu.SemaphoreType.DMA,
            pltpu.SemaphoreType.DMA,
            pltpu.SemaphoreType.DMA((N_CHUNKS,)),
            pltpu.SemaphoreType.DMA((N_CHUNKS,)),
            pltpu.SemaphoreType.DMA((N_CHUNKS,)),
            pltpu.SemaphoreType.DMA((N_CHUNKS,)),
        ],
        input_output_aliases={0: 0},
        compiler_params=pltpu.CompilerParams(collective_id=0),
    )(pltpu.with_memory_space_constraint(x, pltpu.MemorySpace.HBM))


# device time: 19408 ns/iter; 1.3323x vs baseline; 1.3323x over previous
import jax
import jax.numpy as jnp
from jax import lax
from jax.experimental import pallas as pl
from jax.experimental.pallas import tpu as pltpu

CHUNK_ROWS = [32] * 16
N_CHUNKS = len(CHUNK_ROWS)
CHUNK_OFF = [sum(CHUNK_ROWS[:c]) for c in range(N_CHUNKS)]

EXTRA = 32
N_FWD = N_CHUNKS - EXTRA // CHUNK_ROWS[0]


def kernel(x):
    m, n = x.shape
    half = m // 2
    assert sum(CHUNK_ROWS) == half

    def body(x_hbm, out_ref, xv, recv_buf, xv_extra, recv_extra,
             lin_sem, lin2_sem, s1, r1, s2, r2):
        my_x = lax.axis_index("x")
        my_y = lax.axis_index("y")
        x_nbr = (1 - my_x, my_y)
        y_nbr = (my_x, 1 - my_y)

        row0 = my_y * half
        other_row0 = (1 - my_y) * half
        tail0 = other_row0 + half - EXTRA

        lin = pltpu.make_async_copy(
            x_hbm.at[pl.ds(row0, half)], xv, lin_sem
        )
        lin.start()
        lin2 = pltpu.make_async_copy(
            x_hbm.at[pl.ds(tail0, EXTRA)], xv_extra, lin2_sem
        )
        lin2.start()

        barrier_sem = pltpu.get_barrier_semaphore()
        for nbr in (x_nbr, y_nbr):
            pl.semaphore_signal(
                barrier_sem, inc=1,
                device_id=nbr, device_id_type=pl.DeviceIdType.MESH,
            )
        pl.semaphore_wait(barrier_sem, 2)

        rdma1 = []
        for c in range(N_CHUNKS):
            rd = pltpu.make_async_remote_copy(
                src_ref=x_hbm.at[pl.ds(row0 + CHUNK_OFF[c], CHUNK_ROWS[c])],
                dst_ref=recv_buf.at[pl.ds(CHUNK_OFF[c], CHUNK_ROWS[c])],
                send_sem=s1.at[c],
                recv_sem=r1.at[c],
                device_id=x_nbr,
                device_id_type=pl.DeviceIdType.MESH,
            )
            rd.start()
            rdma1.append(rd)

        rdma1e = pltpu.make_async_remote_copy(
            src_ref=x_hbm.at[pl.ds(tail0, EXTRA)],
            dst_ref=recv_extra,
            send_sem=s1.at[N_CHUNKS],
            recv_sem=r1.at[N_CHUNKS],
            device_id=x_nbr,
            device_id_type=pl.DeviceIdType.MESH,
        )
        rdma1e.start()

        lin.wait()

        rdma2 = []
        for c in range(N_CHUNKS):
            rdma1[c].wait_recv()
            rows = pl.ds(CHUNK_OFF[c], CHUNK_ROWS[c])
            out_rows = pl.ds(row0 + CHUNK_OFF[c], CHUNK_ROWS[c])
            out_ref[out_rows, :] = xv[rows, :] + recv_buf[rows, :]
            if c < N_FWD:
                rd = pltpu.make_async_remote_copy(
                    src_ref=out_ref.at[out_rows],
                    dst_ref=out_ref.at[out_rows],
                    send_sem=s2.at[c],
                    recv_sem=r2.at[c],
                    device_id=y_nbr,
                    device_id_type=pl.DeviceIdType.MESH,
                )
                rd.start()
                rdma2.append(rd)

        rdma1e.wait_recv()
        lin2.wait()
        out_ref[pl.ds(tail0, EXTRA), :] = xv_extra[:, :] + recv_extra[:, :]

        for c in range(N_CHUNKS):
            rdma1[c].wait_send()
        rdma1e.wait_send()
        for rd in rdma2:
            rd.wait()

    return pl.pallas_call(
        body,
        out_shape=jax.ShapeDtypeStruct((m, n), x.dtype),
        in_specs=[pl.BlockSpec(memory_space=pltpu.MemorySpace.HBM)],
        out_specs=pl.BlockSpec(memory_space=pltpu.MemorySpace.VMEM),
        scratch_shapes=[
            pltpu.VMEM((half, n), x.dtype),
            pltpu.VMEM((half, n), x.dtype),
            pltpu.VMEM((EXTRA, n), x.dtype),
            pltpu.VMEM((EXTRA, n), x.dtype),
            pltpu.SemaphoreType.DMA,
            pltpu.SemaphoreType.DMA,
            pltpu.SemaphoreType.DMA((N_CHUNKS + 1,)),
            pltpu.SemaphoreType.DMA((N_CHUNKS + 1,)),
            pltpu.SemaphoreType.DMA((N_FWD,)),
            pltpu.SemaphoreType.DMA((N_FWD,)),
        ],
        compiler_params=pltpu.CompilerParams(collective_id=0),
    )(pltpu.with_memory_space_constraint(x, pltpu.MemorySpace.HBM))
